# baseline (speedup 1.0000x reference)
import functools

import jax
import jax.numpy as jnp
from jax import lax
from jax.experimental import pallas as pl
from jax.experimental.pallas import tpu as pltpu
from jax.experimental.pallas import tpu_sc as plsc

_L = 16


@functools.lru_cache(maxsize=None)
def _make_sc_embed(N, S, D):
  try:
    info = plsc.get_sparse_core_info()
    NC, NS = info.num_cores, info.num_subcores
  except ValueError:
    NC, NS = 2, 16
  NW = NC * NS

  assert D == 64 and N % (128 * NW) == 0 and N // 128 == NW
  BT = 128
  DT, DR = D // 8, 8
  n_chunks = S

  mesh = plsc.VectorSubcoreMesh(
      core_axis_name="c", subcore_axis_name="s",
      num_cores=NC, num_subcores=NS)

  @functools.partial(
      pl.kernel,
      mesh=mesh,
      out_type=jax.ShapeDtypeStruct((S, DT, NW, DR, BT), jnp.float32),
      scratch_types=[
          pltpu.VMEM((S, BT), jnp.int32),
          pltpu.VMEM((S * D,), jnp.float32),
          [pltpu.VMEM((BT, 2 * D), jnp.float32) for _ in range(2)],
          [pltpu.VMEM((DT, DR, BT), jnp.float32) for _ in range(2)],
          [pltpu.SemaphoreType.DMA for _ in range(2)],
          [pltpu.SemaphoreType.DMA for _ in range(2)],
      ],
      compiler_params=pltpu.CompilerParams(use_tc_tiling_on_sc=True,
                                           needs_layout_passes=False),
  )
  def sc_embed(wt_hbm, xt_hbm, pos_hbm, out_hbm,
               idx_v, pos_v, gbuf, obuf, gsem, osem):
    wid = lax.axis_index("s") * NC + lax.axis_index("c")

    pltpu.sync_copy(xt_hbm.at[:, pl.ds(wid * BT, BT)], idx_v)
    pltpu.sync_copy(pos_hbm, pos_v)

    def gather(s, b):
      return pltpu.make_async_copy(wt_hbm.at[idx_v.at[s]], gbuf[b], gsem[b])

    def out_copy(s, b):
      return pltpu.make_async_copy(obuf[b], out_hbm.at[s, :, wid], osem[b])

    lane = lax.iota(jnp.int32, _L)
    dt_base = lane // 8
    dr_vec = lane % 8

    gather(0, 0).start()
    gather(1, 1).start()

    @pl.loop(0, n_chunks, step=2)
    def _chunks(s0):
      for b in range(2):
        s = s0 + b
        gather(s, b).wait()
        @pl.when(s - 2 >= 0)
        def _():
          out_copy(s - 2, b).wait()
        p0 = pos_v[pl.ds(s * D, _L)]
        p1 = pos_v[pl.ds(s * D + _L, _L)]
        p2 = pos_v[pl.ds(s * D + 2 * _L, _L)]
        p3 = pos_v[pl.ds(s * D + 3 * _L, _L)]
        pk = (p0, p1, p2, p3)
        @pl.loop(0, BT)
        def _tok(br):
          br_vec = jnp.full((_L,), br, jnp.int32)
          for k in range(D // _L):
            v = gbuf[b][br, pl.ds(k * _L, _L)] + pk[k]
            plsc.store_scatter(obuf[b], [dt_base + 2 * k, dr_vec, br_vec], v)
        out_copy(s, b).start()
        @pl.when(s + 2 < n_chunks)
        def _():
          gather(s + 2, b).start()
    out_copy(n_chunks - 2, 0).wait()
    out_copy(n_chunks - 1, 1).wait()

  return sc_embed


def kernel(x, word_table, pos_table):
  N, S = x.shape
  V, D = word_table.shape
  wt_pad = jnp.pad(word_table, ((0, 0), (0, 128 - D)))
  xt = x.T
  posf = pos_table.reshape(S * D)
  sc = _make_sc_embed(N, S, D)
  out5 = sc(wt_pad, xt, posf)
  return out5.transpose(2, 4, 0, 1, 3).reshape(N, S, D)


# speedup vs baseline: 1.1271x; 1.1271x over previous
import functools

import jax
import jax.numpy as jnp
from jax import lax
from jax.experimental import pallas as pl
from jax.experimental.pallas import tpu as pltpu
from jax.experimental.pallas import tpu_sc as plsc

_L = 16


@functools.lru_cache(maxsize=None)
def _make_sc_embed(N, S, D):
  try:
    info = plsc.get_sparse_core_info()
    NC, NS = info.num_cores, info.num_subcores
  except ValueError:
    NC, NS = 2, 16
  NW = NC * NS

  assert D == 64 and N % (128 * NW) == 0 and N // 128 == NW
  BT = 128
  DT, DR = D // 8, 8
  n_chunks = S

  mesh = plsc.VectorSubcoreMesh(
      core_axis_name="c", subcore_axis_name="s",
      num_cores=NC, num_subcores=NS)

  @functools.partial(
      pl.kernel,
      mesh=mesh,
      out_type=jax.ShapeDtypeStruct((S, DT, NW, DR, BT), jnp.float32),
      scratch_types=[
          pltpu.VMEM((S, BT), jnp.int32),
          pltpu.VMEM((S * D,), jnp.float32),
          [pltpu.VMEM((BT, 2 * D), jnp.float32) for _ in range(4)],
          [pltpu.VMEM((DT, DR, BT), jnp.float32) for _ in range(2)],
          [pltpu.SemaphoreType.DMA for _ in range(4)],
          [pltpu.SemaphoreType.DMA for _ in range(2)],
      ],
      compiler_params=pltpu.CompilerParams(use_tc_tiling_on_sc=True,
                                           needs_layout_passes=False),
  )
  def sc_embed(wt_hbm, xt_hbm, pos_hbm, out_hbm,
               idx_v, pos_v, gbuf, obuf, gsem, osem):
    wid = lax.axis_index("s") * NC + lax.axis_index("c")

    pltpu.sync_copy(xt_hbm.at[:, pl.ds(wid * BT, BT)], idx_v)
    pltpu.sync_copy(pos_hbm, pos_v)

    def gather(s, b):
      return pltpu.make_async_copy(wt_hbm.at[idx_v.at[s]], gbuf[b], gsem[b])

    def out_copy(s, b):
      return pltpu.make_async_copy(obuf[b], out_hbm.at[s, :, wid], osem[b])

    lane = lax.iota(jnp.int32, _L)
    row_vecs = [lane + 16 * j for j in range(BT // _L)]

    gather(0, 0).start()
    gather(1, 1).start()

    @pl.loop(0, n_chunks, step=4)
    def _chunks(s0):
      for i in range(4):
        s = s0 + i
        gb, ob = i % 4, i % 2
        gather(s, gb).wait()
        @pl.when(s + 2 < n_chunks)
        def _():
          gather(s + 2, (i + 2) % 4).start()
        @pl.when(s - 2 >= 0)
        def _():
          out_copy(s - 2, ob).wait()
        @plsc.parallel_loop(0, D // _L, step=1)
        def _feat16(kk):
          pk = pos_v[pl.ds(s * D + kk * _L, _L)]
          for t in range(_L):
            dd = kk * _L + t
            pvec = jnp.full((_L,), pk[t], jnp.float32)
            col = jnp.full((_L,), dd, jnp.int32)
            vals = [plsc.load_gather(gbuf[gb], [row_vecs[j], col]) + pvec
                    for j in range(BT // _L)]
            for j in range(BT // _L):
              obuf[ob][2 * kk + t // DR, t % DR, pl.ds(j * _L, _L)] = vals[j]
        out_copy(s, ob).start()
    out_copy(n_chunks - 2, n_chunks % 2).wait()
    out_copy(n_chunks - 1, (n_chunks + 1) % 2).wait()

  return sc_embed


def kernel(x, word_table, pos_table):
  N, S = x.shape
  V, D = word_table.shape
  wt_pad = jnp.pad(word_table, ((0, 0), (0, 128 - D)))
  xt = x.T
  posf = pos_table.reshape(S * D)
  sc = _make_sc_embed(N, S, D)
  out5 = sc(wt_pad, xt, posf)
  return out5.transpose(2, 4, 0, 1, 3).reshape(N, S, D)


# speedup vs baseline: 1.5431x; 1.3691x over previous
import functools

import jax
import jax.numpy as jnp
from jax import lax
from jax.experimental import pallas as pl
from jax.experimental.pallas import tpu as pltpu
from jax.experimental.pallas import tpu_sc as plsc

_L = 16


@functools.lru_cache(maxsize=None)
def _make_sc_embed(N, S, D):
  try:
    info = plsc.get_sparse_core_info()
    NC, NS = info.num_cores, info.num_subcores
  except ValueError:
    NC, NS = 2, 16
  NW = NC * NS

  assert D == 64 and N % (128 * NW) == 0 and N // 128 == NW
  BT = 128
  DT, DR = D // 8, 8
  n_chunks = S

  mesh = plsc.VectorSubcoreMesh(
      core_axis_name="c", subcore_axis_name="s",
      num_cores=NC, num_subcores=NS)

  @functools.partial(
      pl.kernel,
      mesh=mesh,
      out_type=jax.ShapeDtypeStruct((S, DT, NW, DR, BT), jnp.float32),
      scratch_types=[
          pltpu.VMEM((S, BT), jnp.int32),
          pltpu.VMEM((S * D,), jnp.float32),
          [pltpu.VMEM((BT, 2 * D), jnp.float32) for _ in range(3)],
          [pltpu.VMEM((DT, DR, BT), jnp.float32) for _ in range(2)],
          [pltpu.SemaphoreType.DMA for _ in range(3)],
          [pltpu.SemaphoreType.DMA for _ in range(2)],
      ],
      compiler_params=pltpu.CompilerParams(use_tc_tiling_on_sc=True,
                                           needs_layout_passes=False),
  )
  def sc_embed(wt_hbm, xt_hbm, pos_hbm, out_hbm,
               idx_v, pos_v, gbuf, obuf, gsem, osem):
    wid = lax.axis_index("s") * NC + lax.axis_index("c")

    pltpu.sync_copy(xt_hbm.at[:, pl.ds(wid * BT, BT)], idx_v)
    pltpu.sync_copy(pos_hbm, pos_v)

    def gather(s, b):
      return pltpu.make_async_copy(wt_hbm.at[idx_v.at[s]], gbuf[b], gsem[b])

    def out_copy(s, b):
      return pltpu.make_async_copy(obuf[b], out_hbm.at[s, :, wid], osem[b])

    lane = lax.iota(jnp.int32, _L)

    gather(0, 0).start()
    gather(1, 1).start()

    def chunk_step(s, gb, ob, tail):
      gather(s, gb).wait()
      if not tail:
        gather(s + 2, (gb + 2) % 3).start()
      @pl.when(s - 2 >= 0)
      def _():
        out_copy(s - 2, ob).wait()
      @pl.loop(0, D, unroll=2)
      def _feat(dd):
        f = (lane + dd) % _L
        colv = (dd // _L) * _L + f
        pvec = plsc.load_gather(pos_v, [colv + s * D])
        dtv = f // DR + 2 * (dd // _L)
        drv = f % DR
        for j in range(BT // _L):
          rowv = lane + 16 * j
          v = plsc.load_gather(gbuf[gb], [rowv, colv]) + pvec
          plsc.store_scatter(obuf[ob], [dtv, drv, rowv], v)
      out_copy(s, ob).start()

    @pl.loop(0, n_chunks - 2, step=6)
    def _chunks(s0):
      for i in range(6):
        chunk_step(s0 + i, i % 3, i % 2, tail=False)
    chunk_step(n_chunks - 2, (n_chunks - 2) % 3, (n_chunks - 2) % 2, True)
    chunk_step(n_chunks - 1, (n_chunks - 1) % 3, (n_chunks - 1) % 2, True)
    out_copy(n_chunks - 2, n_chunks % 2).wait()
    out_copy(n_chunks - 1, (n_chunks + 1) % 2).wait()

  return sc_embed


def kernel(x, word_table, pos_table):
  N, S = x.shape
  V, D = word_table.shape
  wt_pad = jnp.pad(word_table, ((0, 0), (0, 128 - D)))
  xt = x.T
  posf = pos_table.reshape(S * D)
  sc = _make_sc_embed(N, S, D)
  out5 = sc(wt_pad, xt, posf)
  return out5.transpose(2, 4, 0, 1, 3).reshape(N, S, D)
